# baseline (speedup 1.0000x reference)
import jax
import jax.numpy as jnp
from jax.experimental import pallas as pl
from jax.experimental.pallas import tpu as pltpu

_TM = 512
_SLOTS = 4


def _g_kernel(w_ref, a_ref, b_ref, out_hbm, scratch, sems):
    i = pl.program_id(0)
    nsteps = pl.num_programs(0)
    result = jnp.dot(a_ref[...] * w_ref[...], b_ref[...],
                     preferred_element_type=jnp.float32)

    for k in range(_SLOTS):
        @pl.when(jax.lax.rem(i, _SLOTS) == k)
        def _(k=k):
            @pl.when(i >= _SLOTS)
            def _():
                pltpu.make_async_copy(
                    scratch.at[k],
                    out_hbm.at[pl.ds((i - _SLOTS) * _TM, _TM), :],
                    sems.at[k],
                ).wait()

            scratch[k] = result
            pltpu.make_async_copy(
                scratch.at[k],
                out_hbm.at[pl.ds(i * _TM, _TM), :],
                sems.at[k],
            ).start()

    @pl.when(i == nsteps - 1)
    def _():
        for k in range(_SLOTS):
            step = nsteps - _SLOTS + k
            slot = step % _SLOTS
            pltpu.make_async_copy(
                scratch.at[slot],
                out_hbm.at[pl.ds(step * _TM, _TM), :],
                sems.at[slot],
            ).wait()


def kernel(DV2_H, invDE_HT_DV2, W):
    N, E = DV2_H.shape
    w2d = W.reshape(1, E)
    return pl.pallas_call(
        _g_kernel,
        grid=(N // _TM,),
        in_specs=[
            pl.BlockSpec((1, E), lambda i: (0, 0)),
            pl.BlockSpec((_TM, E), lambda i: (i, 0)),
            pl.BlockSpec((E, N), lambda i: (0, 0)),
        ],
        out_specs=pl.BlockSpec(memory_space=pl.ANY),
        out_shape=jax.ShapeDtypeStruct((N, N), jnp.float32),
        scratch_shapes=[
            pltpu.VMEM((_SLOTS, _TM, N), jnp.float32),
            pltpu.SemaphoreType.DMA((_SLOTS,)),
        ],
    )(w2d, DV2_H, invDE_HT_DV2)


# speedup vs baseline: 1.0421x; 1.0421x over previous
import jax
import jax.numpy as jnp
from jax.experimental import pallas as pl
from jax.experimental.pallas import tpu as pltpu


def _g_kernel(w_ref, a_ref, b_ref, out_ref):
    a = a_ref[...] * w_ref[...]
    out_ref[...] = jnp.dot(a, b_ref[...], preferred_element_type=jnp.float32)


def kernel(DV2_H, invDE_HT_DV2, W):
    N, E = DV2_H.shape
    TM = 512
    w2d = W.reshape(1, E)
    return pl.pallas_call(
        _g_kernel,
        grid=(N // TM,),
        in_specs=[
            pl.BlockSpec((1, E), lambda i: (0, 0)),
            pl.BlockSpec((TM, E), lambda i: (i, 0)),
            pl.BlockSpec((E, N), lambda i: (0, 0)),
        ],
        out_specs=pl.BlockSpec((TM, N), lambda i: (i, 0)),
        out_shape=jax.ShapeDtypeStruct((N, N), jnp.float32),
        compiler_params=pltpu.CompilerParams(
            dimension_semantics=("parallel",),
        ),
    )(w2d, DV2_H, invDE_HT_DV2)
